# baseline (speedup 1.0000x reference)
import functools

import jax
import jax.numpy as jnp
from jax import lax
from jax.experimental import pallas as pl
from jax.experimental.pallas import tpu as pltpu
from jax.experimental.pallas import tpu_sc as plsc

_N = 100000
_NW = 32
_LPW = 3128
_G = 196
_ROWS = 2 * _LPW


def _sc_body(q_hbm, pt_hbm, outb_hbm, outa_hbm,
             buf0, buf1, pbuf, obv, oav, sem0, sem1):
    wid = lax.axis_index("s") * 2 + lax.axis_index("c")
    base = jnp.minimum(wid * _LPW, _N - _LPW)
    qbase = 2 * base
    pltpu.sync_copy(pt_hbm.at[pl.ds(2 * base, 2 * _LPW)], pbuf)

    iota = lax.broadcasted_iota(jnp.int32, (16,), 0)
    zeros = jnp.zeros((16,), jnp.int32)
    ones = zeros + 1

    def rb(g):
        return jnp.minimum(g * 32, _ROWS - 32)

    def start(g, buf, sem):
        pltpu.make_async_copy(
            q_hbm.at[pl.ds(qbase + rb(g), 32)], buf, sem).start()

    def wait(g, buf, sem):
        pltpu.make_async_copy(
            q_hbm.at[pl.ds(qbase + rb(g), 32)], buf, sem).wait()

    start(0, buf0, sem0)
    start(1, buf1, sem1)

    def outer(o, carry):
        for b in range(2):
            g = o * 2 + b
            buf = (buf0, buf1)[b]
            sem = (sem0, sem1)[b]
            wait(g, buf, sem)
            tl = jnp.minimum(g * 16 + iota, _LPW - 1)
            rowx = 2 * tl - rb(g)
            rowy = rowx + 1
            px = plsc.load_gather(pbuf, [2 * tl])
            py = plsc.load_gather(pbuf, [2 * tl + 1])

            def inner(i, c):
                dmin, imin, ci = c
                x = plsc.load_gather(buf, [rowx, ci])
                y = plsc.load_gather(buf, [rowy, ci])
                dx = x - px
                dy = y - py
                dd = dx * dx + dy * dy
                bt = dd < dmin
                return (jnp.where(bt, dd, dmin),
                        jnp.where(bt, ci, imin),
                        ci + 1)

            dmin0 = jnp.full((16,), jnp.inf, jnp.float32)
            dmin, imin, _ = lax.fori_loop(
                1, 127, inner, (dmin0, zeros, ones), unroll=7)

            cm = imin
            xc = plsc.load_gather(buf, [rowx, cm])
            yc = plsc.load_gather(buf, [rowy, cm])
            xp = plsc.load_gather(buf, [rowx, cm - 1])
            yp = plsc.load_gather(buf, [rowy, cm - 1])
            xn = plsc.load_gather(buf, [rowx, cm + 1])
            yn = plsc.load_gather(buf, [rowy, cm + 1])
            dxp = xp - xc
            dyp = yp - yc
            dxn = xn - xc
            dyn = yn - yc
            dp = dxp * dxp + dyp * dyp
            dn = dxn * dxn + dyn * dyn
            bef = cm - jnp.where(dn < dp, 0, 1)
            plsc.store_scatter(obv, [tl], bef)
            plsc.store_scatter(oav, [tl], bef + 1)

            @pl.when(g + 2 < _G)
            def _():
                start(g + 2, buf, sem)
        return carry

    lax.fori_loop(0, _G // 2, outer, 0)

    pltpu.sync_copy(obv, outb_hbm.at[pl.ds(base, _LPW)])
    pltpu.sync_copy(oav, outa_hbm.at[pl.ds(base, _LPW)])


@jax.jit
def _run_sc(q, pt):
    f = pl.kernel(
        _sc_body,
        out_type=[
            jax.ShapeDtypeStruct((_N,), jnp.int32),
            jax.ShapeDtypeStruct((_N,), jnp.int32),
        ],
        mesh=plsc.VectorSubcoreMesh(core_axis_name="c", subcore_axis_name="s"),
        compiler_params=pltpu.CompilerParams(needs_layout_passes=False),
        scratch_types=[
            pltpu.VMEM((32, 128), jnp.float32),
            pltpu.VMEM((32, 128), jnp.float32),
            pltpu.VMEM((2 * _LPW,), jnp.float32),
            pltpu.VMEM((_LPW,), jnp.int32),
            pltpu.VMEM((_LPW,), jnp.int32),
            pltpu.SemaphoreType.DMA,
            pltpu.SemaphoreType.DMA,
        ],
    )
    return f(q, pt)


def kernel(line_nodes, point):
    n = point.shape[0]
    q = line_nodes.transpose(0, 2, 1).reshape(2 * n, 128)
    ob, oa = _run_sc(q, point.reshape(2 * n))
    return ob, oa


# speedup vs baseline: 1.0001x; 1.0001x over previous
import functools

import jax
import jax.numpy as jnp
from jax import lax
from jax.experimental import pallas as pl
from jax.experimental.pallas import tpu as pltpu
from jax.experimental.pallas import tpu_sc as plsc

_N = 100000
_NW = 32
_LPW = 3128
_G = 196
_ROWS = 2 * _LPW


def _sc_body(q_hbm, pt_hbm, outb_hbm, outa_hbm,
             buf0, buf1, buf2, buf3, pbuf, obv, oav,
             sem0, sem1, sem2, sem3):
    wid = lax.axis_index("s") * 2 + lax.axis_index("c")
    base = jnp.minimum(wid * _LPW, _N - _LPW)
    qbase = 2 * base
    pltpu.sync_copy(pt_hbm.at[pl.ds(2 * base, 2 * _LPW)], pbuf)

    iota = lax.broadcasted_iota(jnp.int32, (16,), 0)
    zeros = jnp.zeros((16,), jnp.int32)
    ones = zeros + 1

    def rb(g):
        return jnp.minimum(g * 32, _ROWS - 32)

    def start(g, buf, sem):
        pltpu.make_async_copy(
            q_hbm.at[pl.ds(qbase + rb(g), 32)], buf, sem).start()

    def wait(g, buf, sem):
        pltpu.make_async_copy(
            q_hbm.at[pl.ds(qbase + rb(g), 32)], buf, sem).wait()

    bufs = (buf0, buf1, buf2, buf3)
    sems = (sem0, sem1, sem2, sem3)
    for b in range(4):
        start(b, bufs[b], sems[b])

    def outer(o, carry):
        for b in range(4):
            g = o * 4 + b
            buf = bufs[b]
            sem = sems[b]
            wait(g, buf, sem)
            tl = jnp.minimum(g * 16 + iota, _LPW - 1)
            rowx = 2 * tl - rb(g)
            rowy = rowx + 1
            px = plsc.load_gather(pbuf, [2 * tl])
            py = plsc.load_gather(pbuf, [2 * tl + 1])

            def inner(i, c):
                dmin, imin, ci = c
                x = plsc.load_gather(buf, [rowx, ci])
                y = plsc.load_gather(buf, [rowy, ci])
                dx = x - px
                dy = y - py
                dd = dx * dx + dy * dy
                bt = dd < dmin
                return (jnp.where(bt, dd, dmin),
                        jnp.where(bt, ci, imin),
                        ci + 1)

            dmin0 = jnp.full((16,), jnp.inf, jnp.float32)
            dmin, imin, _ = lax.fori_loop(
                1, 127, inner, (dmin0, zeros, ones), unroll=7)

            cm = imin
            xc = plsc.load_gather(buf, [rowx, cm])
            yc = plsc.load_gather(buf, [rowy, cm])
            xp = plsc.load_gather(buf, [rowx, cm - 1])
            yp = plsc.load_gather(buf, [rowy, cm - 1])
            xn = plsc.load_gather(buf, [rowx, cm + 1])
            yn = plsc.load_gather(buf, [rowy, cm + 1])
            dxp = xp - xc
            dyp = yp - yc
            dxn = xn - xc
            dyn = yn - yc
            dp = dxp * dxp + dyp * dyp
            dn = dxn * dxn + dyn * dyn
            bef = cm - jnp.where(dn < dp, 0, 1)
            plsc.store_scatter(obv, [tl], bef)
            plsc.store_scatter(oav, [tl], bef + 1)

            @pl.when(g + 4 < _G)
            def _():
                start(g + 4, buf, sem)
        return carry

    lax.fori_loop(0, _G // 4, outer, 0)

    pltpu.sync_copy(obv, outb_hbm.at[pl.ds(base, _LPW)])
    pltpu.sync_copy(oav, outa_hbm.at[pl.ds(base, _LPW)])


@jax.jit
def _run_sc(q, pt):
    f = pl.kernel(
        _sc_body,
        out_type=[
            jax.ShapeDtypeStruct((_N,), jnp.int32),
            jax.ShapeDtypeStruct((_N,), jnp.int32),
        ],
        mesh=plsc.VectorSubcoreMesh(core_axis_name="c", subcore_axis_name="s"),
        compiler_params=pltpu.CompilerParams(needs_layout_passes=False),
        scratch_types=[
            pltpu.VMEM((32, 128), jnp.float32),
            pltpu.VMEM((32, 128), jnp.float32),
            pltpu.VMEM((32, 128), jnp.float32),
            pltpu.VMEM((32, 128), jnp.float32),
            pltpu.VMEM((2 * _LPW,), jnp.float32),
            pltpu.VMEM((_LPW,), jnp.int32),
            pltpu.VMEM((_LPW,), jnp.int32),
            pltpu.SemaphoreType.DMA,
            pltpu.SemaphoreType.DMA,
            pltpu.SemaphoreType.DMA,
            pltpu.SemaphoreType.DMA,
        ],
    )
    return f(q, pt)


def kernel(line_nodes, point):
    n = point.shape[0]
    q = line_nodes.transpose(0, 2, 1).reshape(2 * n, 128)
    ob, oa = _run_sc(q, point.reshape(2 * n))
    return ob, oa
